# baseline (speedup 1.0000x reference)
import functools

import jax
import jax.numpy as jnp
from jax import lax
from jax.experimental import pallas as pl
from jax.experimental.pallas import tpu as pltpu
from jax.experimental.pallas import tpu_sc as plsc

N = 1048576
K = 64
NW = 32
CHUNK = N // NW
L = 16
CH = 2048

NB0 = 1024
NB1 = 1024
NB2 = 2048

_f32 = jnp.float32
_i32 = jnp.int32


def _wid():
    return lax.axis_index("c") * 16 + lax.axis_index("s")


def _zero_hist(hc_ref, hs_ref, nbins):
    @plsc.parallel_loop(0, L * nbins, L, unroll=4)
    def _(i):
        hc_ref[pl.ds(i, L)] = jnp.zeros((L,), _i32)
        hs_ref[pl.ds(i, L)] = jnp.zeros((L,), _f32)


def _batch_copies(pairs, sem):
    cps = [pltpu.make_async_copy(s, d, sem) for s, d in pairs]
    for c in cps:
        c.start()
    for c in cps:
        c.wait()


def _collapse(hc_ref, hs_ref, tc2_ref, ts2_ref, nbins):
    @plsc.parallel_loop(0, nbins, L, unroll=2)
    def _(coff):
        ac = hc_ref[pl.ds(coff, L)]
        asm = hs_ref[pl.ds(coff, L)]
        for s in range(1, L):
            ac = ac + hc_ref[pl.ds(s * nbins + coff, L)]
            asm = asm + hs_ref[pl.ds(s * nbins + coff, L)]
        c = lax.shift_right_logical(coff, 4)
        tc2_ref[c] = ac
        ts2_ref[c] = asm


def _core_merge(tc2_v, ts2_v, shc, shs, idx_ref, cnt_hbm, sum_hbm, nbins):
    core = lax.axis_index("c")
    sid = lax.axis_index("s")
    nrows = nbins // L

    @plsc.parallel_loop(0, nrows, L, unroll=4)
    def _(i):
        idx_ref[pl.ds(i, L)] = lax.iota(_i32, L) + i

    @pl.when(sid == 0)
    def _():
        pltpu.sync_copy(tc2_v, shc)
        pltpu.sync_copy(ts2_v, shs)

    plsc.subcore_barrier()

    @pl.when(sid != 0)
    def _():
        pltpu.sync_copy(tc2_v, shc.at[idx_ref], add=True)
        pltpu.sync_copy(ts2_v, shs.at[idx_ref], add=True)

    plsc.subcore_barrier()

    @pl.when(sid == 0)
    def _():
        pltpu.sync_copy(shc, cnt_hbm.at[core])
        pltpu.sync_copy(shs, sum_hbm.at[core])


def _process(slc2_ref, sls2_ref, f_ref, g_ref, r, c_hi, s_hi, nbins):
    lane = lax.iota(_i32, L)
    nch = nbins // L
    f_ref[pl.ds(nbins, L)] = jnp.zeros((L,), _i32)
    g_ref[pl.ds(nbins, L)] = jnp.zeros((L,), _f32)

    def sbody(j, car):
        car_f, car_g = car
        c = nch - 1 - j
        x = slc2_ref[0, c] + slc2_ref[1, c]
        xs = sls2_ref[0, c] + sls2_ref[1, c]
        fc = lax.rev(plsc.cumsum(lax.rev(x, (0,))), (0,)) + car_f
        gc = lax.rev(plsc.cumsum(lax.rev(xs, (0,))), (0,)) + car_g
        f_ref[pl.ds(c * L, L)] = fc
        g_ref[pl.ds(c * L, L)] = gc
        return (jnp.max(fc), jnp.max(gc))

    lax.fori_loop(0, nch, sbody, (jnp.int32(0), jnp.float32(0.0)))

    def bbody(c, bstar):
        fc = f_ref[pl.ds(c * L, L)]
        cand = jnp.where(fc >= r, lane + c * L, jnp.int32(-1))
        return jnp.maximum(bstar, jnp.max(cand))

    bstar = lax.fori_loop(0, nch, bbody, jnp.int32(-1))

    idxv = jnp.zeros((L,), _i32) + (bstar + 1)
    above_c = jnp.max(plsc.load_gather(f_ref, [idxv]))
    above_s = jnp.max(plsc.load_gather(g_ref, [idxv]))
    return bstar, r - above_c, c_hi + above_c, s_hi + above_s


def _write_state(sti_ref, stf_ref, r, p, c_hi, s_hi):
    zi = jnp.zeros((L,), _i32)
    sti_ref[pl.ds(0, L)] = zi + r
    sti_ref[pl.ds(L, L)] = zi + p
    sti_ref[pl.ds(2 * L, L)] = zi + c_hi
    sti_ref[pl.ds(3 * L, L)] = zi
    stf_ref[pl.ds(0, L)] = jnp.zeros((L,), _f32) + s_hi


def _read_state(sti_ref, stf_ref):
    r = jnp.max(sti_ref[pl.ds(0, L)])
    p = jnp.max(sti_ref[pl.ds(L, L)])
    c_hi = jnp.max(sti_ref[pl.ds(2 * L, L)])
    s_hi = jnp.max(stf_ref[pl.ds(0, L)])
    return r, p, c_hi, s_hi


@functools.lru_cache(maxsize=None)
def _build():
    mesh = plsc.VectorSubcoreMesh(
        core_axis_name="c", subcore_axis_name="s",
        num_cores=2, num_subcores=16)
    cparams = pltpu.CompilerParams(needs_layout_passes=False, use_tc_tiling_on_sc=False)

    def slot_types(nbins):
        return (jax.ShapeDtypeStruct((2, nbins // L, L), _i32),
                jax.ShapeDtypeStruct((2, nbins // L, L), _f32))

    def merge_scratch(nbins):
        return [
            pltpu.VMEM((nbins // L, L), _i32),
            pltpu.VMEM((nbins // L, L), _f32),
            pltpu.VMEM_SHARED((nbins // L, L), _i32),
            pltpu.VMEM_SHARED((nbins // L, L), _f32),
            pltpu.VMEM((nbins // L,), _i32),
        ]

    @functools.partial(
        pl.kernel,
        out_type=(
            jax.ShapeDtypeStruct((N,), _f32),
            *slot_types(NB0),
        ),
        mesh=mesh,
        compiler_params=cparams,
        scratch_types=[
            pltpu.VMEM((CHUNK,), _f32),
            pltpu.VMEM((CHUNK,), _f32),
            pltpu.VMEM((L * NB0,), _i32),
            pltpu.VMEM((L * NB0,), _f32),
            *merge_scratch(NB0),
            pltpu.SemaphoreType.DMA,
        ],
    )
    def k_first(yh_hbm, y_hbm, d2_hbm, cnt_hbm, sum_hbm,
                yh_v, y_v, hc_v, hs_v, tc2_v, ts2_v, shc, shs, midx_v, sem):
        w = _wid()
        base = w * CHUNK
        _batch_copies([
            (yh_hbm.at[pl.ds(base, CHUNK)], yh_v),
            (y_hbm.at[pl.ds(base, CHUNK)], y_v),
        ], sem)

        _zero_hist(hc_v, hs_v, NB0)
        lane = lax.iota(_i32, L)
        ones = jnp.ones((L,), _i32)

        @plsc.parallel_loop(0, CHUNK, L, unroll=8)
        def _(i):
            a = yh_v[pl.ds(i, L)]
            b = y_v[pl.ds(i, L)]
            d = a - b
            d2v = d * d
            yh_v[pl.ds(i, L)] = d2v
            bits = lax.bitcast_convert_type(d2v, _i32)
            binv = lax.shift_right_logical(bits, 21)
            idx = lane * NB0 + binv
            plsc.addupdate_scatter(hc_v, [idx], ones)
            plsc.addupdate_scatter(hs_v, [idx], d2v)

        d2cp = pltpu.make_async_copy(yh_v, d2_hbm.at[pl.ds(base, CHUNK)], sem)
        d2cp.start()
        _collapse(hc_v, hs_v, tc2_v, ts2_v, NB0)
        _core_merge(tc2_v, ts2_v, shc, shs, midx_v, cnt_hbm, sum_hbm, NB0)
        d2cp.wait()

    def make_mid(first, prev_width, mask_shift, bin_shift, bin_mask,
                 nb_prev, nb, writes_cand, zero_dma=False):
        outs = [
            *slot_types(nb),
            jax.ShapeDtypeStruct((4 * L,), _i32),
            jax.ShapeDtypeStruct((L,), _f32),
        ]
        scratch = [
            pltpu.VMEM((CHUNK if first else CH,), _f32),
            pltpu.VMEM((2, nb_prev // L, L), _i32),
            pltpu.VMEM((2, nb_prev // L, L), _f32),
            pltpu.VMEM((L * nb,), _i32),
            pltpu.VMEM((L * nb,), _f32),
            pltpu.VMEM((nb_prev + L,), _i32),
            pltpu.VMEM((nb_prev + L,), _f32),
            *merge_scratch(nb),
            pltpu.VMEM((4 * L,), _i32),
            pltpu.VMEM((L,), _f32),
            pltpu.VMEM((L,), _i32),
            pltpu.SemaphoreType.DMA,
        ]
        if writes_cand:
            outs += [
                jax.ShapeDtypeStruct((N,), _f32),
                jax.ShapeDtypeStruct((NW * L,), _i32),
            ]
            scratch += [
                pltpu.VMEM((CHUNK,), _f32),
            ]

        @functools.partial(
            pl.kernel,
            out_type=tuple(outs),
            mesh=mesh,
            compiler_params=cparams,
            scratch_types=scratch,
        )
        def k_mid(*args):
            it = iter(args)
            dat_hbm = next(it)
            if not first:
                ccnt_in_hbm = next(it)
            pcnt_hbm = next(it)
            psum_hbm = next(it)
            if not first:
                sti_hbm = next(it)
                stf_hbm = next(it)
            if zero_dma:
                zc_hbm = next(it)
                zs_hbm = next(it)
            cnt_hbm = next(it)
            sum_hbm = next(it)
            sto_i_hbm = next(it)
            sto_f_hbm = next(it)
            if writes_cand:
                cand_hbm = next(it)
                ccnt_hbm = next(it)
            dat_v = next(it)
            slc_v = next(it)
            sls_v = next(it)
            hc_v = next(it)
            hs_v = next(it)
            f_v = next(it)
            g_v = next(it)
            tc2_v = next(it)
            ts2_v = next(it)
            shc = next(it)
            shs = next(it)
            midx_v = next(it)
            sti_v = next(it)
            stf_v = next(it)
            crow_v = next(it)
            sem = next(it)
            if writes_cand:
                cout_v = next(it)

            w = _wid()
            base = w * CHUNK
            copies = [(pcnt_hbm, slc_v), (psum_hbm, sls_v)]
            if first:
                copies.append((dat_hbm.at[pl.ds(base, CHUNK)], dat_v))
            else:
                copies += [(sti_hbm, sti_v), (stf_hbm, stf_v),
                           (ccnt_in_hbm.at[pl.ds(w * L, L)], crow_v)]
            if zero_dma:
                copies += [(zc_hbm, hc_v), (zs_hbm, hs_v)]
            _batch_copies(copies, sem)
            if first:
                r0 = jnp.int32(K)
                p0 = jnp.int32(0)
                c0 = jnp.int32(0)
                s0 = jnp.float32(0.0)
                n_in = jnp.int32(CHUNK)
            else:
                r0, p0, c0, s0 = _read_state(sti_v, stf_v)
                n_in = jnp.max(crow_v[pl.ds(0, L)])
            bstar, r1, c1, s1 = _process(slc_v, sls_v, f_v, g_v,
                                         r0, c0, s0, nb_prev)
            if first:
                p1 = bstar
            else:
                p1 = jnp.bitwise_or(lax.shift_left(p0, prev_width), bstar)

            if not zero_dma:
                _zero_hist(hc_v, hs_v, nb)
            lane = lax.iota(_i32, L)
            ones = jnp.ones((L,), _i32)
            pv = jnp.zeros((L,), _i32) + p1

            def sweep_body(i, goff, pos):
                d2v = dat_v[pl.ds(i, L)]
                bits = lax.bitcast_convert_type(d2v, _i32)
                m = lax.shift_right_logical(bits, mask_shift) == pv
                if not first:
                    m = jnp.logical_and(m, (lane + goff) < n_in)
                binv = jnp.bitwise_and(
                    lax.shift_right_logical(bits, bin_shift),
                    jnp.int32(bin_mask))
                idx = lane * nb + binv
                plsc.addupdate_scatter(hc_v, [idx], ones, mask=m)
                plsc.addupdate_scatter(hs_v, [idx], d2v, mask=m)
                if writes_cand:
                    plsc.store_compressed(cout_v.at[pl.ds(pos, L)], d2v,
                                          mask=m)
                    pos = pos + plsc.all_reduce_population_count(m)[0]
                return pos

            if first:
                @plsc.parallel_loop(0, CHUNK, L, unroll=8,
                                    carry=jnp.int32(0))
                def n_out(i, pos):
                    return sweep_body(i, i, pos)
            else:
                nch = (n_in + (CH - 1)) // CH

                def chunk_loop(ci, pos):
                    off = ci * CH
                    pltpu.sync_copy(dat_hbm.at[pl.ds(base + off, CH)], dat_v)
                    rem = n_in - off
                    m16 = (jnp.minimum(jnp.int32(CH), rem) + (L - 1)) // L
                    pos = lax.fori_loop(
                        0, m16,
                        lambda j, pp: sweep_body(j * L, off + j * L, pp),
                        pos)
                    return pos

                n_out = lax.fori_loop(0, nch, chunk_loop, jnp.int32(0))

            _collapse(hc_v, hs_v, tc2_v, ts2_v, nb)
            _core_merge(tc2_v, ts2_v, shc, shs, midx_v, cnt_hbm, sum_hbm, nb)

            if writes_cand:
                crow_v[pl.ds(0, L)] = jnp.zeros((L,), _i32) + n_out
                pltpu.sync_copy(crow_v, ccnt_hbm.at[pl.ds(w * L, L)])
                ncho = (n_out + (CH - 1)) // CH

                def dma_out(ci, _):
                    off = ci * CH
                    pltpu.sync_copy(cout_v.at[pl.ds(off, CH)],
                                    cand_hbm.at[pl.ds(base + off, CH)])
                    return 0

                lax.fori_loop(0, ncho, dma_out, 0)

            @pl.when(w == 0)
            def _():
                _write_state(sti_v, stf_v, r1, p1, c1, s1)
                pltpu.sync_copy(sti_v, sto_i_hbm)
                pltpu.sync_copy(stf_v, sto_f_hbm)

        return k_mid

    @functools.partial(
        pl.kernel,
        out_type=jax.ShapeDtypeStruct((L,), _f32),
        mesh=mesh,
        compiler_params=cparams,
        scratch_types=[
            pltpu.VMEM((2, NB2 // L, L), _i32),
            pltpu.VMEM((2, NB2 // L, L), _f32),
            pltpu.VMEM((NB2 + L,), _i32),
            pltpu.VMEM((NB2 + L,), _f32),
            pltpu.VMEM((4 * L,), _i32),
            pltpu.VMEM((L,), _f32),
            pltpu.VMEM((L,), _f32),
            pltpu.SemaphoreType.DMA,
        ],
    )
    def k_last(pcnt_hbm, psum_hbm, sti_hbm, stf_hbm, out_hbm,
               slc_v, sls_v, f_v, g_v, sti_v, stf_v, out_v, sem):
        w = _wid()
        _batch_copies([
            (pcnt_hbm, slc_v), (psum_hbm, sls_v),
            (sti_hbm, sti_v), (stf_hbm, stf_v),
        ], sem)
        r0, p0, c0, s0 = _read_state(sti_v, stf_v)
        bstar, r1, c1, s1 = _process(slc_v, sls_v, f_v, g_v,
                                     r0, c0, s0, NB2)
        t_bits = jnp.bitwise_or(lax.shift_left(p0, 11), bstar)
        tv = lax.bitcast_convert_type(jnp.zeros((L,), _i32) + t_bits, _f32)
        t = jnp.max(tv)
        ans = (s1 + lax.convert_element_type(r1, _f32) * t) * (1.0 / K)

        @pl.when(w == 0)
        def _():
            out_v[pl.ds(0, L)] = jnp.zeros((L,), _f32) + ans
            pltpu.sync_copy(out_v, out_hbm)

    k_mid1 = make_mid(True, 10, 21, 11, 0x3FF, NB0, NB1, True)
    k_mid2 = make_mid(False, 10, 11, 0, 0x7FF, NB1, NB2, False,
                      zero_dma=True)
    return k_first, k_mid1, k_mid2, k_last


def kernel(yhat, y):
    k_first, k_mid1, k_mid2, k_last = _build()
    zc = jnp.zeros((L * NB2,), _i32)
    zs = jnp.zeros((L * NB2,), _f32)
    d2, c0, s0 = k_first(yhat, y)
    c1, s1, i1, f1, cand1, cc1 = k_mid1(d2, c0, s0)
    c2, s2, i2, f2 = k_mid2(cand1, cc1, c1, s1, i1, f1, zc, zs)
    out = k_last(c2, s2, i2, f2)
    return out[0]


# speedup vs baseline: 1.0390x; 1.0390x over previous
import functools

import jax
import jax.numpy as jnp
from jax import lax
from jax.experimental import pallas as pl
from jax.experimental.pallas import tpu as pltpu
from jax.experimental.pallas import tpu_sc as plsc

N = 1048576
K = 64
NW = 32
CHUNK = N // NW
L = 16
CH = 2048

NB0 = 1024
NB1 = 1024
NB2 = 2048

_f32 = jnp.float32
_i32 = jnp.int32


def _wid():
    return lax.axis_index("c") * 16 + lax.axis_index("s")


def _zero_hist(hc_ref, hs_ref, nbins):
    @plsc.parallel_loop(0, L * nbins, L, unroll=4)
    def _(i):
        hc_ref[pl.ds(i, L)] = jnp.zeros((L,), _i32)
        hs_ref[pl.ds(i, L)] = jnp.zeros((L,), _f32)


def _batch_copies(pairs, sem):
    cps = [pltpu.make_async_copy(s, d, sem) for s, d in pairs]
    for c in cps:
        c.start()
    for c in cps:
        c.wait()


def _collapse(hc_ref, hs_ref, tc2_ref, ts2_ref, nbins):
    @plsc.parallel_loop(0, nbins, L, unroll=2)
    def _(coff):
        ac = hc_ref[pl.ds(coff, L)]
        asm = hs_ref[pl.ds(coff, L)]
        for s in range(1, L):
            ac = ac + hc_ref[pl.ds(s * nbins + coff, L)]
            asm = asm + hs_ref[pl.ds(s * nbins + coff, L)]
        c = lax.shift_right_logical(coff, 4)
        tc2_ref[c] = ac
        ts2_ref[c] = asm


def _core_merge(tc2_v, ts2_v, shc, shs, idx_ref, cnt_hbm, sum_hbm, nbins):
    core = lax.axis_index("c")
    sid = lax.axis_index("s")
    nrows = nbins // L

    @plsc.parallel_loop(0, nrows, L, unroll=4)
    def _(i):
        idx_ref[pl.ds(i, L)] = lax.iota(_i32, L) + i

    @pl.when(sid == 0)
    def _():
        pltpu.sync_copy(tc2_v, shc)
        pltpu.sync_copy(ts2_v, shs)

    plsc.subcore_barrier()

    @pl.when(sid != 0)
    def _():
        pltpu.sync_copy(tc2_v, shc.at[idx_ref], add=True)
        pltpu.sync_copy(ts2_v, shs.at[idx_ref], add=True)

    plsc.subcore_barrier()

    @pl.when(sid == 0)
    def _():
        pltpu.sync_copy(shc, cnt_hbm.at[core])
        pltpu.sync_copy(shs, sum_hbm.at[core])


def _process(slc2_ref, sls2_ref, f_ref, g_ref, r, c_hi, s_hi, nbins):
    lane = lax.iota(_i32, L)
    nch = nbins // L
    f_ref[pl.ds(nbins, L)] = jnp.zeros((L,), _i32)
    g_ref[pl.ds(nbins, L)] = jnp.zeros((L,), _f32)

    def sbody(j, car):
        car_f, car_g = car
        c = nch - 1 - j
        x = slc2_ref[0, c] + slc2_ref[1, c]
        xs = sls2_ref[0, c] + sls2_ref[1, c]
        fc = lax.rev(plsc.cumsum(lax.rev(x, (0,))), (0,)) + car_f
        gc = lax.rev(plsc.cumsum(lax.rev(xs, (0,))), (0,)) + car_g
        f_ref[pl.ds(c * L, L)] = fc
        g_ref[pl.ds(c * L, L)] = gc
        return (jnp.max(fc), jnp.max(gc))

    lax.fori_loop(0, nch, sbody, (jnp.int32(0), jnp.float32(0.0)))

    def bbody(c, bstar):
        fc = f_ref[pl.ds(c * L, L)]
        cand = jnp.where(fc >= r, lane + c * L, jnp.int32(-1))
        return jnp.maximum(bstar, jnp.max(cand))

    bstar = lax.fori_loop(0, nch, bbody, jnp.int32(-1))

    idxv = jnp.zeros((L,), _i32) + (bstar + 1)
    above_c = jnp.max(plsc.load_gather(f_ref, [idxv]))
    above_s = jnp.max(plsc.load_gather(g_ref, [idxv]))
    return bstar, r - above_c, c_hi + above_c, s_hi + above_s


def _write_state(sti_ref, stf_ref, r, p, c_hi, s_hi):
    zi = jnp.zeros((L,), _i32)
    sti_ref[pl.ds(0, L)] = zi + r
    sti_ref[pl.ds(L, L)] = zi + p
    sti_ref[pl.ds(2 * L, L)] = zi + c_hi
    sti_ref[pl.ds(3 * L, L)] = zi
    stf_ref[pl.ds(0, L)] = jnp.zeros((L,), _f32) + s_hi


def _read_state(sti_ref, stf_ref):
    r = jnp.max(sti_ref[pl.ds(0, L)])
    p = jnp.max(sti_ref[pl.ds(L, L)])
    c_hi = jnp.max(sti_ref[pl.ds(2 * L, L)])
    s_hi = jnp.max(stf_ref[pl.ds(0, L)])
    return r, p, c_hi, s_hi


@functools.lru_cache(maxsize=None)
def _build():
    mesh = plsc.VectorSubcoreMesh(
        core_axis_name="c", subcore_axis_name="s",
        num_cores=2, num_subcores=16)
    cparams = pltpu.CompilerParams(needs_layout_passes=False, use_tc_tiling_on_sc=False)

    def slot_types(nbins):
        return (jax.ShapeDtypeStruct((2, nbins // L, L), _i32),
                jax.ShapeDtypeStruct((2, nbins // L, L), _f32))

    def merge_scratch(nbins):
        return [
            pltpu.VMEM((nbins // L, L), _i32),
            pltpu.VMEM((nbins // L, L), _f32),
            pltpu.VMEM_SHARED((nbins // L, L), _i32),
            pltpu.VMEM_SHARED((nbins // L, L), _f32),
            pltpu.VMEM((nbins // L,), _i32),
        ]

    @functools.partial(
        pl.kernel,
        out_type=(
            jax.ShapeDtypeStruct((N,), _f32),
            *slot_types(NB0),
        ),
        mesh=mesh,
        compiler_params=cparams,
        scratch_types=[
            pltpu.VMEM((CHUNK,), _f32),
            pltpu.VMEM((CHUNK,), _f32),
            pltpu.VMEM((L * NB0,), _i32),
            pltpu.VMEM((L * NB0,), _f32),
            *merge_scratch(NB0),
            pltpu.SemaphoreType.DMA,
        ],
    )
    def k_first(yh_hbm, y_hbm, d2_hbm, cnt_hbm, sum_hbm,
                yh_v, y_v, hc_v, hs_v, tc2_v, ts2_v, shc, shs, midx_v, sem):
        w = _wid()
        base = w * CHUNK
        _batch_copies([
            (yh_hbm.at[pl.ds(base, CHUNK)], yh_v),
            (y_hbm.at[pl.ds(base, CHUNK)], y_v),
        ], sem)

        _zero_hist(hc_v, hs_v, NB0)
        lane = lax.iota(_i32, L)
        ones = jnp.ones((L,), _i32)

        @plsc.parallel_loop(0, CHUNK, L, unroll=8)
        def _(i):
            a = yh_v[pl.ds(i, L)]
            b = y_v[pl.ds(i, L)]
            d = a - b
            d2v = d * d
            yh_v[pl.ds(i, L)] = d2v
            bits = lax.bitcast_convert_type(d2v, _i32)
            binv = lax.shift_right_logical(bits, 21)
            idx = lane * NB0 + binv
            plsc.addupdate_scatter(hc_v, [idx], ones)
            plsc.addupdate_scatter(hs_v, [idx], d2v)

        d2cp = pltpu.make_async_copy(yh_v, d2_hbm.at[pl.ds(base, CHUNK)], sem)
        d2cp.start()
        _collapse(hc_v, hs_v, tc2_v, ts2_v, NB0)
        _core_merge(tc2_v, ts2_v, shc, shs, midx_v, cnt_hbm, sum_hbm, NB0)
        d2cp.wait()

    def make_mid(first, prev_width, mask_shift, bin_shift, bin_mask,
                 nb_prev, nb, writes_cand, zero_dma=False):
        outs = [
            *slot_types(nb),
            jax.ShapeDtypeStruct((4 * L,), _i32),
            jax.ShapeDtypeStruct((L,), _f32),
        ]
        scratch = [
            pltpu.VMEM((CHUNK if first else CH,), _f32),
            pltpu.VMEM((2, nb_prev // L, L), _i32),
            pltpu.VMEM((2, nb_prev // L, L), _f32),
            pltpu.VMEM((L * nb,), _i32),
            pltpu.VMEM((L * nb,), _f32),
            pltpu.VMEM((nb_prev + L,), _i32),
            pltpu.VMEM((nb_prev + L,), _f32),
            *merge_scratch(nb),
            pltpu.VMEM((4 * L,), _i32),
            pltpu.VMEM((L,), _f32),
            pltpu.VMEM((L,), _i32),
            pltpu.SemaphoreType.DMA,
        ]
        if writes_cand:
            outs += [
                jax.ShapeDtypeStruct((N,), _f32),
                jax.ShapeDtypeStruct((NW * L,), _i32),
            ]
            scratch += [
                pltpu.VMEM((CHUNK,), _f32),
            ]

        @functools.partial(
            pl.kernel,
            out_type=tuple(outs),
            mesh=mesh,
            compiler_params=cparams,
            scratch_types=scratch,
        )
        def k_mid(*args):
            it = iter(args)
            dat_hbm = next(it)
            if not first:
                ccnt_in_hbm = next(it)
            pcnt_hbm = next(it)
            psum_hbm = next(it)
            if not first:
                sti_hbm = next(it)
                stf_hbm = next(it)
            if zero_dma:
                zc_hbm = next(it)
                zs_hbm = next(it)
            cnt_hbm = next(it)
            sum_hbm = next(it)
            sto_i_hbm = next(it)
            sto_f_hbm = next(it)
            if writes_cand:
                cand_hbm = next(it)
                ccnt_hbm = next(it)
            dat_v = next(it)
            slc_v = next(it)
            sls_v = next(it)
            hc_v = next(it)
            hs_v = next(it)
            f_v = next(it)
            g_v = next(it)
            tc2_v = next(it)
            ts2_v = next(it)
            shc = next(it)
            shs = next(it)
            midx_v = next(it)
            sti_v = next(it)
            stf_v = next(it)
            crow_v = next(it)
            sem = next(it)
            if writes_cand:
                cout_v = next(it)

            w = _wid()
            base = w * CHUNK
            copies = [(pcnt_hbm, slc_v), (psum_hbm, sls_v)]
            if first:
                copies.append((dat_hbm.at[pl.ds(base, CHUNK)], dat_v))
            else:
                copies += [(sti_hbm, sti_v), (stf_hbm, stf_v),
                           (ccnt_in_hbm.at[pl.ds(w * L, L)], crow_v)]
            if zero_dma:
                copies += [(zc_hbm, hc_v), (zs_hbm, hs_v)]
            _batch_copies(copies, sem)
            if first:
                r0 = jnp.int32(K)
                p0 = jnp.int32(0)
                c0 = jnp.int32(0)
                s0 = jnp.float32(0.0)
                n_in = jnp.int32(CHUNK)
            else:
                r0, p0, c0, s0 = _read_state(sti_v, stf_v)
                n_in = jnp.max(crow_v[pl.ds(0, L)])
            bstar, r1, c1, s1 = _process(slc_v, sls_v, f_v, g_v,
                                         r0, c0, s0, nb_prev)
            if first:
                p1 = bstar
            else:
                p1 = jnp.bitwise_or(lax.shift_left(p0, prev_width), bstar)

            if not zero_dma:
                _zero_hist(hc_v, hs_v, nb)
            lane = lax.iota(_i32, L)
            ones = jnp.ones((L,), _i32)
            pv = jnp.zeros((L,), _i32) + p1

            def sweep_body(i, goff, pos):
                d2v = dat_v[pl.ds(i, L)]
                bits = lax.bitcast_convert_type(d2v, _i32)
                m = lax.shift_right_logical(bits, mask_shift) == pv
                if not first:
                    m = jnp.logical_and(m, (lane + goff) < n_in)
                binv = jnp.bitwise_and(
                    lax.shift_right_logical(bits, bin_shift),
                    jnp.int32(bin_mask))
                idx = lane * nb + binv
                plsc.addupdate_scatter(hc_v, [idx], ones, mask=m)
                plsc.addupdate_scatter(hs_v, [idx], d2v, mask=m)
                if writes_cand:
                    plsc.store_compressed(cout_v.at[pl.ds(pos, L)], d2v,
                                          mask=m)
                    pos = pos + plsc.all_reduce_population_count(m)[0]
                return pos

            if first:
                @plsc.parallel_loop(0, CHUNK, L, unroll=8,
                                    carry=jnp.int32(0))
                def n_out(i, pos):
                    return sweep_body(i, i, pos)
            else:
                nch = (n_in + (CH - 1)) // CH

                def chunk_loop(ci, pos):
                    off = ci * CH
                    pltpu.sync_copy(dat_hbm.at[pl.ds(base + off, CH)], dat_v)
                    rem = n_in - off
                    m16 = (jnp.minimum(jnp.int32(CH), rem) + (L - 1)) // L
                    pos = lax.fori_loop(
                        0, m16,
                        lambda j, pp: sweep_body(j * L, off + j * L, pp),
                        pos)
                    return pos

                n_out = lax.fori_loop(0, nch, chunk_loop, jnp.int32(0))

            _collapse(hc_v, hs_v, tc2_v, ts2_v, nb)
            _core_merge(tc2_v, ts2_v, shc, shs, midx_v, cnt_hbm, sum_hbm, nb)

            if writes_cand:
                crow_v[pl.ds(0, L)] = jnp.zeros((L,), _i32) + n_out
                pltpu.sync_copy(crow_v, ccnt_hbm.at[pl.ds(w * L, L)])
                ncho = (n_out + (CH - 1)) // CH

                def dma_out(ci, _):
                    off = ci * CH
                    pltpu.sync_copy(cout_v.at[pl.ds(off, CH)],
                                    cand_hbm.at[pl.ds(base + off, CH)])
                    return 0

                lax.fori_loop(0, ncho, dma_out, 0)

            @pl.when(w == 0)
            def _():
                _write_state(sti_v, stf_v, r1, p1, c1, s1)
                pltpu.sync_copy(sti_v, sto_i_hbm)
                pltpu.sync_copy(stf_v, sto_f_hbm)

        return k_mid

    @functools.partial(
        pl.kernel,
        out_type=jax.ShapeDtypeStruct((L,), _f32),
        mesh=mesh,
        compiler_params=cparams,
        scratch_types=[
            pltpu.VMEM((2, NB2 // L, L), _i32),
            pltpu.VMEM((2, NB2 // L, L), _f32),
            pltpu.VMEM((NB2 + L,), _i32),
            pltpu.VMEM((NB2 + L,), _f32),
            pltpu.VMEM((4 * L,), _i32),
            pltpu.VMEM((L,), _f32),
            pltpu.VMEM((L,), _f32),
            pltpu.SemaphoreType.DMA,
        ],
    )
    def k_last(pcnt_hbm, psum_hbm, sti_hbm, stf_hbm, out_hbm,
               slc_v, sls_v, f_v, g_v, sti_v, stf_v, out_v, sem):
        w = _wid()
        _batch_copies([
            (pcnt_hbm, slc_v), (psum_hbm, sls_v),
            (sti_hbm, sti_v), (stf_hbm, stf_v),
        ], sem)
        r0, p0, c0, s0 = _read_state(sti_v, stf_v)
        bstar, r1, c1, s1 = _process(slc_v, sls_v, f_v, g_v,
                                     r0, c0, s0, NB2)
        t_bits = jnp.bitwise_or(lax.shift_left(p0, 11), bstar)
        tv = lax.bitcast_convert_type(jnp.zeros((L,), _i32) + t_bits, _f32)
        t = jnp.max(tv)
        ans = (s1 + lax.convert_element_type(r1, _f32) * t) * (1.0 / K)

        @pl.when(w == 0)
        def _():
            out_v[pl.ds(0, L)] = jnp.zeros((L,), _f32) + ans
            pltpu.sync_copy(out_v, out_hbm)

    k_mid1 = make_mid(True, 10, 21, 11, 0x3FF, NB0, NB1, True)
    k_mid2 = make_mid(False, 10, 11, 0, 0x7FF, NB1, NB2, False,
                      zero_dma=False)
    return k_first, k_mid1, k_mid2, k_last


def kernel(yhat, y):
    k_first, k_mid1, k_mid2, k_last = _build()
    d2, c0, s0 = k_first(yhat, y)
    c1, s1, i1, f1, cand1, cc1 = k_mid1(d2, c0, s0)
    c2, s2, i2, f2 = k_mid2(cand1, cc1, c1, s1, i1, f1)
    out = k_last(c2, s2, i2, f2)
    return out[0]
